# baseline (speedup 1.0000x reference)
import functools

import numpy as np
import jax
import jax.numpy as jnp
from jax.experimental import pallas as pl
from jax.experimental.pallas import tpu as pltpu

_TT = 1024


@functools.lru_cache(maxsize=None)
def _mask_constants(B, T):
    rng = np.random.RandomState(0)
    r = 0.0
    perm = np.zeros(0, dtype=np.int64)
    for _ in range(B):
        r = rng.rand()
        perm = rng.permutation(T - 10)
    rank = np.full((T,), np.iinfo(np.int32).max, dtype=np.int32)
    rank[10 + perm] = np.arange(T - 10, dtype=np.int32)
    return rank, float(r)


def _mask_kernel(nm_ref, x_ref, rank_ref, emb_ref, o_ref):
    b = pl.program_id(1)
    nm = nm_ref[0]
    rank = rank_ref[...]
    masked = (rank < nm) & (b == 0)
    x = x_ref[0]
    emb = emb_ref[...]
    o_ref[0] = jnp.where(masked, emb, x)


def kernel(tensor, padding_mask, mask_prob, mask_length, min_masks, mask_emb):
    B, T, D = tensor.shape
    rank_np, r = _mask_constants(B, T)
    rank = jnp.asarray(rank_np).reshape(T, 1)

    mp = jnp.reshape(mask_prob, (-1,))[0]
    seq_len = T - jnp.sum(padding_mask[B - 1])
    num_mask = jnp.maximum(
        min_masks,
        jnp.floor(mp * seq_len / mask_length + r).astype(jnp.int32),
    ).astype(jnp.int32)

    grid = (T // _TT, B)
    out = pl.pallas_call(
        _mask_kernel,
        grid_spec=pltpu.PrefetchScalarGridSpec(
            num_scalar_prefetch=1,
            grid=grid,
            in_specs=[
                pl.BlockSpec((1, _TT, D), lambda j, b, nm: (b, j, 0)),
                pl.BlockSpec((_TT, 1), lambda j, b, nm: (j, 0)),
                pl.BlockSpec((1, D), lambda j, b, nm: (0, 0)),
            ],
            out_specs=pl.BlockSpec((1, _TT, D), lambda j, b, nm: (b, j, 0)),
        ),
        compiler_params=pltpu.CompilerParams(
            dimension_semantics=("parallel", "parallel"),
        ),
        out_shape=jax.ShapeDtypeStruct((B, T, D), tensor.dtype),
    )(num_mask.reshape(1), tensor, rank, mask_emb.reshape(1, D))
    return out


# speedup vs baseline: 1.0169x; 1.0169x over previous
import functools

import numpy as np
import jax
import jax.numpy as jnp
from jax.experimental import pallas as pl
from jax.experimental.pallas import tpu as pltpu

_TT = 2048


@functools.lru_cache(maxsize=None)
def _mask_constants(B, T):
    rng = np.random.RandomState(0)
    r = 0.0
    perm = np.zeros(0, dtype=np.int64)
    for _ in range(B):
        r = rng.rand()
        perm = rng.permutation(T - 10)
    rank = np.full((T,), np.iinfo(np.int32).max, dtype=np.int32)
    rank[10 + perm] = np.arange(T - 10, dtype=np.int32)
    return rank, float(r)


def _mask_kernel(nm_ref, x_ref, rank_ref, emb_ref, o_ref):
    b = pl.program_id(1)
    nm = nm_ref[0]
    rank = rank_ref[...]
    masked = (rank < nm) & (b == 0)
    x = x_ref[0]
    emb = emb_ref[...]
    o_ref[0] = jnp.where(masked, emb, x)


def kernel(tensor, padding_mask, mask_prob, mask_length, min_masks, mask_emb):
    B, T, D = tensor.shape
    rank_np, r = _mask_constants(B, T)
    rank = jnp.asarray(rank_np).reshape(T, 1)

    mp = jnp.reshape(mask_prob, (-1,))[0]
    seq_len = T - jnp.sum(padding_mask[B - 1])
    num_mask = jnp.maximum(
        min_masks,
        jnp.floor(mp * seq_len / mask_length + r).astype(jnp.int32),
    ).astype(jnp.int32)

    grid = (T // _TT, B)
    out = pl.pallas_call(
        _mask_kernel,
        grid_spec=pltpu.PrefetchScalarGridSpec(
            num_scalar_prefetch=1,
            grid=grid,
            in_specs=[
                pl.BlockSpec((1, _TT, D), lambda j, b, nm: (b, j, 0)),
                pl.BlockSpec((_TT, 1), lambda j, b, nm: (j, 0)),
                pl.BlockSpec((1, D), lambda j, b, nm: (0, 0)),
            ],
            out_specs=pl.BlockSpec((1, _TT, D), lambda j, b, nm: (b, j, 0)),
        ),
        compiler_params=pltpu.CompilerParams(
            dimension_semantics=("parallel", "parallel"),
        ),
        out_shape=jax.ShapeDtypeStruct((B, T, D), tensor.dtype),
    )(num_mask.reshape(1), tensor, rank, mask_emb.reshape(1, D))
    return out


# speedup vs baseline: 1.0190x; 1.0021x over previous
import functools

import numpy as np
import jax
import jax.numpy as jnp
from jax.experimental import pallas as pl
from jax.experimental.pallas import tpu as pltpu

_TT = 2048


@functools.lru_cache(maxsize=None)
def _mask_constants(B, T):
    rng = np.random.RandomState(0)
    r = 0.0
    perm = np.zeros(0, dtype=np.int64)
    for _ in range(B):
        r = rng.rand()
        perm = rng.permutation(T - 10)
    rank = np.full((T,), np.iinfo(np.int32).max, dtype=np.int32)
    rank[10 + perm] = np.arange(T - 10, dtype=np.int32)
    return rank, float(r)


def _mask_kernel(nm_ref, x_ref, rank_ref, emb_ref, o_ref):
    b = pl.program_id(1)
    nm = nm_ref[0]
    rank = rank_ref[...]
    masked = (rank < nm) & (b == 0)
    x = x_ref[0]
    emb = emb_ref[...]
    o_ref[0] = jnp.where(masked, emb, x)


def kernel(tensor, padding_mask, mask_prob, mask_length, min_masks, mask_emb):
    B, T, D = tensor.shape
    rank_np, r = _mask_constants(B, T)
    rank = jnp.asarray(rank_np).reshape(T, 1)

    mp = jnp.reshape(mask_prob, (-1,))[0]
    seq_len = T - jnp.sum(padding_mask[B - 1])
    num_mask = jnp.maximum(
        min_masks,
        jnp.floor(mp * seq_len / mask_length + r).astype(jnp.int32),
    ).astype(jnp.int32)

    grid = (T // _TT, B)
    out = pl.pallas_call(
        _mask_kernel,
        grid_spec=pltpu.PrefetchScalarGridSpec(
            num_scalar_prefetch=1,
            grid=grid,
            in_specs=[
                pl.BlockSpec((1, _TT, D), lambda j, b, nm: (b, j, 0)),
                pl.BlockSpec((_TT, 1), lambda j, b, nm: (j, 0)),
                pl.BlockSpec((1, D), lambda j, b, nm: (0, 0)),
            ],
            out_specs=pl.BlockSpec((1, _TT, D), lambda j, b, nm: (b, j, 0)),
        ),
        compiler_params=pltpu.CompilerParams(
            dimension_semantics=("arbitrary", "arbitrary"),
        ),
        out_shape=jax.ShapeDtypeStruct((B, T, D), tensor.dtype),
    )(num_mask.reshape(1), tensor, rank, mask_emb.reshape(1, D))
    return out
